# Initial kernel scaffold; baseline (speedup 1.0000x reference)
import jax
import jax.numpy as jnp
from jax.experimental import pallas as pl


def kernel(hidden_states, gate_w, Wg, Wu, Wd):
    raise NotImplementedError("write your pallas kernel here")


# speedup vs baseline: 1.2857x; 1.2857x over previous
import functools
import jax
import jax.numpy as jnp
from jax import lax
from jax.experimental import pallas as pl
from jax.experimental.pallas import tpu as pltpu

HIDDEN = 2048
N_EXPERTS = 8
TOP_K = 2
D_FF = 1024
T_BLOCK = 512


def _router_body(x_ref, gw_ref, logits_ref, comb_ref):
    x = x_ref[...]
    gw = gw_ref[...]
    l = lax.dot_general(x, gw, (((1,), (1,)), ((), ())),
                        preferred_element_type=jnp.float32)
    logits_ref[...] = l
    m = jnp.max(l, axis=1, keepdims=True)
    ex = jnp.exp(l - m)
    sm = ex / jnp.sum(ex, axis=1, keepdims=True)
    iota = lax.broadcasted_iota(jnp.int32, sm.shape, 1)
    m1 = jnp.max(sm, axis=1, keepdims=True)
    a1 = jnp.min(jnp.where(sm == m1, iota, N_EXPERTS), axis=1, keepdims=True)
    sm2 = jnp.where(iota == a1, -1.0, sm)
    m2 = jnp.max(sm2, axis=1, keepdims=True)
    a2 = jnp.min(jnp.where(sm2 == m2, iota, N_EXPERTS), axis=1, keepdims=True)
    comb_ref[...] = jnp.where(iota == a1, m1, jnp.where(iota == a2, m2, 0.0))


def _ffn_body(x_ref, comb_ref, wg_ref, wu_ref, wd_ref, out_ref):
    e = pl.program_id(1)

    @pl.when(e == 0)
    def _():
        out_ref[...] = jnp.zeros_like(out_ref)

    x = x_ref[...]
    g = lax.dot_general(x, wg_ref[0], (((1,), (1,)), ((), ())),
                        preferred_element_type=jnp.float32)
    u = lax.dot_general(x, wu_ref[0], (((1,), (1,)), ((), ())),
                        preferred_element_type=jnp.float32)
    h = (g * jax.nn.sigmoid(g)) * u
    y = lax.dot_general(h.astype(jnp.bfloat16), wd_ref[0],
                        (((1,), (1,)), ((), ())),
                        preferred_element_type=jnp.float32)
    comb = comb_ref[...]
    lane = lax.broadcasted_iota(jnp.int32, comb.shape, 1)
    w_e = jnp.sum(jnp.where(lane == e, comb, 0.0), axis=1, keepdims=True)
    out_ref[...] += w_e * y


@jax.jit
def kernel(hidden_states, gate_w, Wg, Wu, Wd):
    B, S, D = hidden_states.shape
    x = hidden_states.reshape(-1, D)
    T = x.shape[0]
    n_tb = T // T_BLOCK

    logits, comb = pl.pallas_call(
        _router_body,
        grid=(n_tb,),
        in_specs=[
            pl.BlockSpec((T_BLOCK, D), lambda t: (t, 0)),
            pl.BlockSpec((N_EXPERTS, D), lambda t: (0, 0)),
        ],
        out_specs=[
            pl.BlockSpec((T_BLOCK, N_EXPERTS), lambda t: (t, 0)),
            pl.BlockSpec((T_BLOCK, N_EXPERTS), lambda t: (t, 0)),
        ],
        out_shape=[
            jax.ShapeDtypeStruct((T, N_EXPERTS), jnp.float32),
            jax.ShapeDtypeStruct((T, N_EXPERTS), jnp.float32),
        ],
    )(x, gate_w)

    xb = x.astype(jnp.bfloat16)
    wg = Wg.astype(jnp.bfloat16)
    wu = Wu.astype(jnp.bfloat16)
    wd = Wd.astype(jnp.bfloat16)

    final = pl.pallas_call(
        _ffn_body,
        grid=(n_tb, N_EXPERTS),
        in_specs=[
            pl.BlockSpec((T_BLOCK, D), lambda t, e: (t, 0)),
            pl.BlockSpec((T_BLOCK, N_EXPERTS), lambda t, e: (t, 0)),
            pl.BlockSpec((1, D_FF, D), lambda t, e: (e, 0, 0)),
            pl.BlockSpec((1, D_FF, D), lambda t, e: (e, 0, 0)),
            pl.BlockSpec((1, D, D_FF), lambda t, e: (e, 0, 0)),
        ],
        out_specs=pl.BlockSpec((T_BLOCK, D), lambda t, e: (t, 0)),
        out_shape=jax.ShapeDtypeStruct((T, D), jnp.float32),
        compiler_params=pltpu.CompilerParams(
            dimension_semantics=("arbitrary", "arbitrary"),
        ),
    )(xb, comb, wg, wu, wd)

    return final.reshape(B, S, D), logits
